# baseline (speedup 1.0000x reference)
import functools

import jax
import jax.numpy as jnp
from jax import lax
from jax.experimental import pallas as pl
from jax.experimental.pallas import tpu as pltpu
from jax.experimental.pallas import tpu_sc as plsc

N = 10000
E = 320000
D = 128
DH = D // 2

NC = 2
NS = 16
CHUNK = 128
NCHUNKS_PAD = 2560
CPT = NCHUNKS_PAD // NS
E_PAD = NCHUNKS_PAD * CHUNK
NBUF = 4
GROUPS = CPT // NBUF
Z_ROWS = N + 8
R_STD = 624
R_LAST = N - (NS - 1) * R_STD


def _pre_body(x_ref, w_ref, b_ref, o_ref):
    t = jnp.maximum(
        jnp.dot(x_ref[...], w_ref[...], preferred_element_type=jnp.float32)
        + b_ref[...], 0.0)
    o_ref[0] = t[:, :DH]
    o_ref[1] = t[:, DH:]


def _update_body(x_ref, z0_ref, z1_ref, w1x_ref, w1za_ref, w1zb_ref, b1_ref,
                 w2_ref, b2_ref, o_ref):
    t = jnp.maximum(
        jnp.dot(x_ref[...], w1x_ref[...], preferred_element_type=jnp.float32)
        + jnp.dot(z0_ref[...], w1za_ref[...],
                  preferred_element_type=jnp.float32)
        + jnp.dot(z1_ref[...], w1zb_ref[...],
                  preferred_element_type=jnp.float32)
        + b1_ref[...], 0.0)
    o_ref[...] = (jnp.dot(t, w2_ref[...], preferred_element_type=jnp.float32)
                  + b2_ref[...])


_mesh = plsc.VectorSubcoreMesh(core_axis_name="c", subcore_axis_name="s")


@functools.partial(
    pl.kernel,
    out_type=jax.ShapeDtypeStruct((NC, N, DH), jnp.float32),
    mesh=_mesh,
    scratch_types=[
        pltpu.VMEM((CPT, CHUNK), jnp.int32),
        pltpu.VMEM((CPT, CHUNK), jnp.int32),
        pltpu.VMEM((CHUNK, DH), jnp.float32),
        pltpu.VMEM((CHUNK, DH), jnp.float32),
        pltpu.VMEM((CHUNK, DH), jnp.float32),
        pltpu.VMEM((CHUNK, DH), jnp.float32),
        pltpu.VMEM_SHARED((Z_ROWS, DH), jnp.float32),
        pltpu.SemaphoreType.DMA,
        pltpu.SemaphoreType.DMA,
        pltpu.SemaphoreType.DMA,
        pltpu.SemaphoreType.DMA,
    ],
    compiler_params=pltpu.CompilerParams(use_tc_tiling_on_sc=False),
)
def _segment_sum_sc(m_hbm, src_hbm, dst_hbm, zeros_hbm, out_hbm,
                    src_v, dst_v, r0, r1, r2, r3, z_sh, s0, s1, s2, s3):
    cid = lax.axis_index("c")
    sid = lax.axis_index("s")
    rows = (r0, r1, r2, r3)
    sems = (s0, s1, s2, s3)
    m_half = m_hbm.at[cid]

    c0 = pl.multiple_of(sid * CPT, 8)
    pltpu.sync_copy(src_hbm.at[pl.ds(c0, CPT)], src_v)
    pltpu.sync_copy(dst_hbm.at[pl.ds(c0, CPT)], dst_v)
    for b in range(NBUF):
        pltpu.async_copy(m_half.at[src_v.at[b]], rows[b], sems[b])

    zr0 = pl.multiple_of(sid * R_STD, 8)

    @pl.when(sid < NS - 1)
    def _():
        pltpu.sync_copy(zeros_hbm.at[pl.ds(0, R_STD)],
                        z_sh.at[pl.ds(zr0, R_STD)])

    @pl.when(sid == NS - 1)
    def _():
        pltpu.sync_copy(zeros_hbm, z_sh.at[pl.ds(zr0, R_LAST)])

    plsc.subcore_barrier()

    def group(g, issue_next):
        for b in range(NBUF):
            i = g * NBUF + b
            pltpu.make_async_copy(m_half.at[src_v.at[i]], rows[b],
                                  sems[b]).wait()
            pltpu.sync_copy(rows[b], z_sh.at[dst_v.at[i]], add=True)
            if issue_next:
                pltpu.async_copy(m_half.at[src_v.at[i + NBUF]], rows[b],
                                 sems[b])

    def body(g, carry):
        group(g, True)
        return carry

    lax.fori_loop(0, GROUPS - 1, body, 0)
    group(GROUPS - 1, False)

    plsc.subcore_barrier()

    @pl.when(sid < NS - 1)
    def _():
        pltpu.sync_copy(z_sh.at[pl.ds(zr0, R_STD)],
                        out_hbm.at[cid, pl.ds(zr0, R_STD)])

    @pl.when(sid == NS - 1)
    def _():
        pltpu.sync_copy(z_sh.at[pl.ds(zr0, R_LAST)],
                        out_hbm.at[cid, pl.ds(zr0, R_LAST)])


def kernel(x, edge_index, W_pre, b_pre, W_u1, b_u1, W_u2, b_u2):
    src = edge_index[0].astype(jnp.int32)
    dst = edge_index[1].astype(jnp.int32)
    pad = E_PAD - E
    src2d = jnp.concatenate(
        [src, jnp.zeros((pad,), jnp.int32)]).reshape(NCHUNKS_PAD, CHUNK)
    dst2d = jnp.concatenate(
        [dst, jnp.full((pad,), N, jnp.int32)]).reshape(NCHUNKS_PAD, CHUNK)

    m = pl.pallas_call(
        _pre_body,
        out_shape=jax.ShapeDtypeStruct((NC, N, DH), jnp.float32),
    )(x, W_pre, b_pre.reshape(1, D))

    zeros = jnp.zeros((R_LAST, DH), dtype=jnp.float32)
    z_parts = _segment_sum_sc(m, src2d, dst2d, zeros)

    h = pl.pallas_call(
        _update_body,
        out_shape=jax.ShapeDtypeStruct((N, D), jnp.float32),
    )(x, z_parts[0], z_parts[1], W_u1[:D], W_u1[D:D + DH], W_u1[D + DH:],
      b_u1.reshape(1, D), W_u2, b_u2.reshape(1, D))
    return h


# speedup vs baseline: 1.1397x; 1.1397x over previous
import functools

import jax
import jax.numpy as jnp
from jax import lax
from jax.experimental import pallas as pl
from jax.experimental.pallas import tpu as pltpu
from jax.experimental.pallas import tpu_sc as plsc

N = 10000
E = 320000
D = 128

NC = 2
NS = 16
NW = NC * NS
CHUNK = 128
NCHUNKS_PAD = 2560
CPT = NCHUNKS_PAD // NW
E_PAD = NCHUNKS_PAD * CHUNK
NBUF = 4
GROUPS = CPT // NBUF
Z_ROWS = N + 8
R_STD = 624
R_LAST = N - (NS - 1) * R_STD


def _pre_body(x_ref, w_ref, b_ref, o_ref):
    o_ref[...] = jnp.maximum(
        jnp.dot(x_ref[...], w_ref[...], preferred_element_type=jnp.float32)
        + b_ref[...], 0.0).astype(jnp.bfloat16)


def _update_body(x_ref, z0_ref, z1_ref, w1x_ref, w1z_ref, b1_ref, w2_ref,
                 b2_ref, o_ref):
    z = z0_ref[...].astype(jnp.float32) + z1_ref[...].astype(jnp.float32)
    t = jnp.maximum(
        jnp.dot(x_ref[...], w1x_ref[...], preferred_element_type=jnp.float32)
        + jnp.dot(z, w1z_ref[...], preferred_element_type=jnp.float32)
        + b1_ref[...], 0.0)
    o_ref[...] = (jnp.dot(t, w2_ref[...], preferred_element_type=jnp.float32)
                  + b2_ref[...])


_mesh = plsc.VectorSubcoreMesh(core_axis_name="c", subcore_axis_name="s")


@functools.partial(
    pl.kernel,
    out_type=jax.ShapeDtypeStruct((NC, N, D), jnp.bfloat16),
    mesh=_mesh,
    scratch_types=[
        pltpu.VMEM((CPT, CHUNK), jnp.int32),
        pltpu.VMEM((CPT, CHUNK), jnp.int32),
        pltpu.VMEM((CHUNK, D), jnp.bfloat16),
        pltpu.VMEM((CHUNK, D), jnp.bfloat16),
        pltpu.VMEM((CHUNK, D), jnp.bfloat16),
        pltpu.VMEM((CHUNK, D), jnp.bfloat16),
        pltpu.VMEM_SHARED((Z_ROWS, D), jnp.bfloat16),
        pltpu.SemaphoreType.DMA,
        pltpu.SemaphoreType.DMA,
        pltpu.SemaphoreType.DMA,
        pltpu.SemaphoreType.DMA,
    ],
    compiler_params=pltpu.CompilerParams(use_tc_tiling_on_sc=False),
)
def _segment_sum_sc(m_hbm, src_hbm, dst_hbm, zeros_hbm, out_hbm,
                    src_v, dst_v, r0, r1, r2, r3, z_sh, s0, s1, s2, s3):
    cid = lax.axis_index("c")
    sid = lax.axis_index("s")
    wid = sid * NC + cid
    rows = (r0, r1, r2, r3)
    sems = (s0, s1, s2, s3)

    c0 = pl.multiple_of(wid * CPT, 8)
    pltpu.sync_copy(src_hbm.at[pl.ds(c0, CPT)], src_v)
    pltpu.sync_copy(dst_hbm.at[pl.ds(c0, CPT)], dst_v)
    for b in range(NBUF):
        pltpu.async_copy(m_hbm.at[src_v.at[b]], rows[b], sems[b])

    zr0 = pl.multiple_of(sid * R_STD, 8)

    @pl.when(sid < NS - 1)
    def _():
        pltpu.sync_copy(zeros_hbm.at[pl.ds(0, R_STD)],
                        z_sh.at[pl.ds(zr0, R_STD)])

    @pl.when(sid == NS - 1)
    def _():
        pltpu.sync_copy(zeros_hbm, z_sh.at[pl.ds(zr0, R_LAST)])

    plsc.subcore_barrier()

    def group(g, issue_next):
        for b in range(NBUF):
            i = g * NBUF + b
            pltpu.make_async_copy(m_hbm.at[src_v.at[i]], rows[b],
                                  sems[b]).wait()
            pltpu.sync_copy(rows[b], z_sh.at[dst_v.at[i]], add=True)
            if issue_next:
                pltpu.async_copy(m_hbm.at[src_v.at[i + NBUF]], rows[b],
                                 sems[b])

    def body(g, carry):
        group(g, True)
        return carry

    lax.fori_loop(0, GROUPS - 1, body, 0)
    group(GROUPS - 1, False)

    plsc.subcore_barrier()

    @pl.when(sid < NS - 1)
    def _():
        pltpu.sync_copy(z_sh.at[pl.ds(zr0, R_STD)],
                        out_hbm.at[cid, pl.ds(zr0, R_STD)])

    @pl.when(sid == NS - 1)
    def _():
        pltpu.sync_copy(z_sh.at[pl.ds(zr0, R_LAST)],
                        out_hbm.at[cid, pl.ds(zr0, R_LAST)])


def kernel(x, edge_index, W_pre, b_pre, W_u1, b_u1, W_u2, b_u2):
    src = edge_index[0].astype(jnp.int32)
    dst = edge_index[1].astype(jnp.int32)
    pad = E_PAD - E
    src2d = jnp.concatenate(
        [src, jnp.zeros((pad,), jnp.int32)]).reshape(NCHUNKS_PAD, CHUNK)
    dst2d = jnp.concatenate(
        [dst, jnp.full((pad,), N, jnp.int32)]).reshape(NCHUNKS_PAD, CHUNK)

    m = pl.pallas_call(
        _pre_body,
        out_shape=jax.ShapeDtypeStruct((N, D), jnp.bfloat16),
    )(x, W_pre, b_pre.reshape(1, D))

    zeros = jnp.zeros((R_LAST, D), dtype=jnp.bfloat16)
    z_parts = _segment_sum_sc(m, src2d, dst2d, zeros)

    h = pl.pallas_call(
        _update_body,
        out_shape=jax.ShapeDtypeStruct((N, D), jnp.float32),
    )(x, z_parts[0], z_parts[1], W_u1[:D], W_u1[D:], b_u1.reshape(1, D),
      W_u2, b_u2.reshape(1, D))
    return h


# speedup vs baseline: 1.7457x; 1.5316x over previous
import functools

import jax
import jax.numpy as jnp
from jax import lax
from jax.experimental import pallas as pl
from jax.experimental.pallas import tpu as pltpu
from jax.experimental.pallas import tpu_sc as plsc

N = 10000
E = 320000
D = 128

NC = 2
NS = 16
NW = NC * NS
CHUNK = 128
NCHUNKS_PAD = 2560
CPT = NCHUNKS_PAD // NW
E_PAD = NCHUNKS_PAD * CHUNK
NBUF = 4
GROUPS = CPT // NBUF
M_ROWS = N + 8
R_STD = 624
R_LAST = N - (NS - 1) * R_STD


def _pre_body(x_ref, w_ref, b_ref, o_ref):
    o_ref[pl.ds(0, N)] = jnp.maximum(
        jnp.dot(x_ref[...], w_ref[...], preferred_element_type=jnp.float32)
        + b_ref[...], 0.0).astype(jnp.bfloat16)
    o_ref[pl.ds(N, M_ROWS - N)] = jnp.zeros((M_ROWS - N, D), jnp.bfloat16)


def _update_body(x_ref, z0_ref, z1_ref, w1x_ref, w1z_ref, b1_ref, w2_ref,
                 b2_ref, o_ref):
    z = z0_ref[...].astype(jnp.float32) + z1_ref[...].astype(jnp.float32)
    t = jnp.maximum(
        jnp.dot(x_ref[...], w1x_ref[...], preferred_element_type=jnp.float32)
        + jnp.dot(z, w1z_ref[...], preferred_element_type=jnp.float32)
        + b1_ref[...], 0.0)
    o_ref[...] = (jnp.dot(t, w2_ref[...], preferred_element_type=jnp.float32)
                  + b2_ref[...])


_mesh = plsc.VectorSubcoreMesh(core_axis_name="c", subcore_axis_name="s")


@functools.partial(
    pl.kernel,
    out_type=jax.ShapeDtypeStruct((NC, N, D), jnp.bfloat16),
    mesh=_mesh,
    scratch_types=[
        pltpu.VMEM((CPT, CHUNK), jnp.int32),
        pltpu.VMEM((CPT, CHUNK), jnp.int32),
        pltpu.VMEM((CHUNK, D), jnp.bfloat16),
        pltpu.VMEM((CHUNK, D), jnp.bfloat16),
        pltpu.VMEM((CHUNK, D), jnp.bfloat16),
        pltpu.VMEM((CHUNK, D), jnp.bfloat16),
        pltpu.VMEM_SHARED((N, D), jnp.bfloat16),
        pltpu.SemaphoreType.DMA,
        pltpu.SemaphoreType.DMA,
        pltpu.SemaphoreType.DMA,
        pltpu.SemaphoreType.DMA,
    ],
    compiler_params=pltpu.CompilerParams(use_tc_tiling_on_sc=False),
)
def _segment_sum_sc(m_hbm, src_hbm, dst_hbm, zeros_hbm, out_hbm,
                    src_v, dst_v, r0, r1, r2, r3, z_sh, s0, s1, s2, s3):
    cid = lax.axis_index("c")
    sid = lax.axis_index("s")
    wid = sid * NC + cid
    rows = (r0, r1, r2, r3)
    sems = (s0, s1, s2, s3)

    c0 = pl.multiple_of(wid * CPT, 8)
    pltpu.sync_copy(src_hbm.at[pl.ds(c0, CPT)], src_v)
    pltpu.sync_copy(dst_hbm.at[pl.ds(c0, CPT)], dst_v)
    for b in range(NBUF):
        pltpu.async_copy(m_hbm.at[src_v.at[b]], rows[b], sems[b])

    zr0 = pl.multiple_of(sid * R_STD, 8)

    @pl.when(sid < NS - 1)
    def _():
        pltpu.sync_copy(zeros_hbm.at[pl.ds(0, R_STD)],
                        z_sh.at[pl.ds(zr0, R_STD)])

    @pl.when(sid == NS - 1)
    def _():
        pltpu.sync_copy(zeros_hbm, z_sh.at[pl.ds(zr0, R_LAST)])

    plsc.subcore_barrier()

    def group(g, issue_next):
        for b in range(NBUF):
            i = g * NBUF + b
            pltpu.make_async_copy(m_hbm.at[src_v.at[i]], rows[b],
                                  sems[b]).wait()
            pltpu.sync_copy(rows[b], z_sh.at[dst_v.at[i]], add=True)
            if issue_next:
                pltpu.async_copy(m_hbm.at[src_v.at[i + NBUF]], rows[b],
                                 sems[b])

    def body(g, carry):
        group(g, True)
        return carry

    lax.fori_loop(0, GROUPS - 1, body, 0)
    group(GROUPS - 1, False)

    plsc.subcore_barrier()

    @pl.when(sid < NS - 1)
    def _():
        pltpu.sync_copy(z_sh.at[pl.ds(zr0, R_STD)],
                        out_hbm.at[cid, pl.ds(zr0, R_STD)])

    @pl.when(sid == NS - 1)
    def _():
        pltpu.sync_copy(z_sh.at[pl.ds(zr0, R_LAST)],
                        out_hbm.at[cid, pl.ds(zr0, R_LAST)])


def kernel(x, edge_index, W_pre, b_pre, W_u1, b_u1, W_u2, b_u2):
    src = edge_index[0].astype(jnp.int32)
    dst = edge_index[1].astype(jnp.int32)
    pad = E_PAD - E
    j = jnp.arange(pad, dtype=jnp.int32)
    src2d = jnp.concatenate(
        [src, N + (j % (M_ROWS - N))]).reshape(NCHUNKS_PAD, CHUNK)
    dst2d = jnp.concatenate(
        [dst, (j * 131) % N]).reshape(NCHUNKS_PAD, CHUNK)

    m = pl.pallas_call(
        _pre_body,
        out_shape=jax.ShapeDtypeStruct((M_ROWS, D), jnp.bfloat16),
    )(x, W_pre, b_pre.reshape(1, D))

    zeros = jnp.zeros((R_LAST, D), dtype=jnp.bfloat16)
    z_parts = _segment_sum_sc(m, src2d, dst2d, zeros)

    h = pl.pallas_call(
        _update_body,
        out_shape=jax.ShapeDtypeStruct((N, D), jnp.float32),
    )(x, z_parts[0], z_parts[1], W_u1[:D], W_u1[D:], b_u1.reshape(1, D),
      W_u2, b_u2.reshape(1, D))
    return h


# speedup vs baseline: 2.6018x; 1.4904x over previous
import functools

import jax
import jax.numpy as jnp
from jax import lax
from jax.experimental import pallas as pl
from jax.experimental.pallas import tpu as pltpu
from jax.experimental.pallas import tpu_sc as plsc

N = 10000
E = 320000
D = 128

NC = 2
NS = 16
NW = NC * NS
CHUNK = 128
NCHUNKS_PAD = 2560
CPT = NCHUNKS_PAD // NW
E_PAD = NCHUNKS_PAD * CHUNK
NBUF = 4
GROUPS = CPT // NBUF
M_ROWS = N + 2048
R_STD = 624
R_LAST = N - (NS - 1) * R_STD


def _pre_body(x_ref, w_ref, b_ref, o_ref):
    o_ref[pl.ds(0, N)] = jnp.maximum(
        jnp.dot(x_ref[...], w_ref[...], preferred_element_type=jnp.float32)
        + b_ref[...], 0.0).astype(jnp.bfloat16)
    o_ref[pl.ds(N, M_ROWS - N)] = jnp.zeros((M_ROWS - N, D), jnp.bfloat16)


def _update_body(x_ref, z0_ref, z1_ref, w1x_ref, w1z_ref, b1_ref, w2_ref,
                 b2_ref, o_ref):
    z = z0_ref[...].astype(jnp.float32) + z1_ref[...].astype(jnp.float32)
    t = jnp.maximum(
        jnp.dot(x_ref[...], w1x_ref[...], preferred_element_type=jnp.float32)
        + jnp.dot(z, w1z_ref[...], preferred_element_type=jnp.float32)
        + b1_ref[...], 0.0)
    o_ref[...] = (jnp.dot(t, w2_ref[...], preferred_element_type=jnp.float32)
                  + b2_ref[...])


_mesh = plsc.VectorSubcoreMesh(core_axis_name="c", subcore_axis_name="s")


@functools.partial(
    pl.kernel,
    out_type=jax.ShapeDtypeStruct((NC, N, D), jnp.bfloat16),
    mesh=_mesh,
    scratch_types=[
        pltpu.VMEM((CPT, CHUNK), jnp.int32),
        pltpu.VMEM((CPT, CHUNK), jnp.int32),
        pltpu.VMEM((CHUNK, D), jnp.bfloat16),
        pltpu.VMEM((CHUNK, D), jnp.bfloat16),
        pltpu.VMEM((CHUNK, D), jnp.bfloat16),
        pltpu.VMEM((CHUNK, D), jnp.bfloat16),
        pltpu.VMEM_SHARED((N, D), jnp.bfloat16),
        pltpu.SemaphoreType.DMA,
        pltpu.SemaphoreType.DMA,
        pltpu.SemaphoreType.DMA,
        pltpu.SemaphoreType.DMA,
    ],
    compiler_params=pltpu.CompilerParams(use_tc_tiling_on_sc=False),
)
def _segment_sum_sc(m_hbm, src_hbm, dst_hbm, zeros_hbm, out_hbm,
                    src_v, dst_v, r0, r1, r2, r3, z_sh, s0, s1, s2, s3):
    cid = lax.axis_index("c")
    sid = lax.axis_index("s")
    wid = sid * NC + cid
    rows = (r0, r1, r2, r3)
    sems = (s0, s1, s2, s3)

    c0 = pl.multiple_of(wid * CPT, 8)
    pltpu.sync_copy(src_hbm.at[pl.ds(c0, CPT)], src_v)
    pltpu.sync_copy(dst_hbm.at[pl.ds(c0, CPT)], dst_v)
    for b in range(NBUF):
        pltpu.async_copy(m_hbm.at[src_v.at[b]], rows[b], sems[b])

    zr0 = pl.multiple_of(sid * R_STD, 8)

    @pl.when(sid < NS - 1)
    def _():
        pltpu.sync_copy(zeros_hbm.at[pl.ds(0, R_STD)],
                        z_sh.at[pl.ds(zr0, R_STD)])

    @pl.when(sid == NS - 1)
    def _():
        pltpu.sync_copy(zeros_hbm, z_sh.at[pl.ds(zr0, R_LAST)])

    plsc.subcore_barrier()

    def group(g, issue_next):
        for b in range(NBUF):
            i = g * NBUF + b
            pltpu.make_async_copy(m_hbm.at[src_v.at[i]], rows[b],
                                  sems[b]).wait()
            pltpu.sync_copy(rows[b], z_sh.at[dst_v.at[i]], add=True)
            if issue_next:
                pltpu.async_copy(m_hbm.at[src_v.at[i + NBUF]], rows[b],
                                 sems[b])

    def body(g, carry):
        group(g, True)
        return carry

    lax.fori_loop(0, GROUPS - 1, body, 0)
    group(GROUPS - 1, False)

    plsc.subcore_barrier()

    @pl.when(sid < NS - 1)
    def _():
        pltpu.sync_copy(z_sh.at[pl.ds(zr0, R_STD)],
                        out_hbm.at[cid, pl.ds(zr0, R_STD)])

    @pl.when(sid == NS - 1)
    def _():
        pltpu.sync_copy(z_sh.at[pl.ds(zr0, R_LAST)],
                        out_hbm.at[cid, pl.ds(zr0, R_LAST)])


def kernel(x, edge_index, W_pre, b_pre, W_u1, b_u1, W_u2, b_u2):
    src = edge_index[0].astype(jnp.int32)
    dst = edge_index[1].astype(jnp.int32)
    pad = E_PAD - E
    j = jnp.arange(pad, dtype=jnp.int32)
    src2d = jnp.concatenate(
        [src, N + (j % (M_ROWS - N))]).reshape(NCHUNKS_PAD, CHUNK)
    dst2d = jnp.concatenate(
        [dst, (j * 131) % N]).reshape(NCHUNKS_PAD, CHUNK)

    m = pl.pallas_call(
        _pre_body,
        out_shape=jax.ShapeDtypeStruct((M_ROWS, D), jnp.bfloat16),
    )(x, W_pre, b_pre.reshape(1, D))

    zeros = jnp.zeros((R_LAST, D), dtype=jnp.bfloat16)
    z_parts = _segment_sum_sc(m, src2d, dst2d, zeros)

    h = pl.pallas_call(
        _update_body,
        out_shape=jax.ShapeDtypeStruct((N, D), jnp.float32),
    )(x, z_parts[0], z_parts[1], W_u1[:D], W_u1[D:], b_u1.reshape(1, D),
      W_u2, b_u2.reshape(1, D))
    return h


# speedup vs baseline: 2.9244x; 1.1240x over previous
import functools

import jax
import jax.numpy as jnp
import numpy as np
from jax import lax
from jax.experimental import pallas as pl
from jax.experimental.pallas import tpu as pltpu
from jax.experimental.pallas import tpu_sc as plsc

N = 10000
E = 320000
D = 128

NC = 2
NS = 16
NW = NC * NS
CHUNK = 128
NCHUNKS = E // CHUNK
CPT = 80
NBUF = 4
GROUPS = CPT // NBUF
LAST = NW - 1
MAIN_N = NCHUNKS - LAST * CPT
PAD_N = CPT - MAIN_N
M_ROWS = N + 2048

_j = np.arange(PAD_N * CHUNK, dtype=np.int32)
_PSRC = (N + (_j % (M_ROWS - N))).reshape(PAD_N, CHUNK)
_PDST = ((_j * 131) % N).reshape(PAD_N, CHUNK)
R_STD = 624
R_LAST = N - (NS - 1) * R_STD


def _pre_body(x_ref, w_ref, b_ref, o_ref):
    o_ref[pl.ds(0, N)] = jnp.maximum(
        jnp.dot(x_ref[...], w_ref[...], preferred_element_type=jnp.float32)
        + b_ref[...], 0.0).astype(jnp.bfloat16)
    o_ref[pl.ds(N, M_ROWS - N)] = jnp.zeros((M_ROWS - N, D), jnp.bfloat16)


_NCHUNKS_ALIGNED = LAST * CPT


def _update_body(x_ref, z0_ref, z1_ref, w1x_ref, w1z_ref, b1_ref, w2_ref,
                 b2_ref, o_ref):
    z = z0_ref[...].astype(jnp.float32) + z1_ref[...].astype(jnp.float32)
    t = jnp.maximum(
        jnp.dot(x_ref[...], w1x_ref[...], preferred_element_type=jnp.float32)
        + jnp.dot(z, w1z_ref[...], preferred_element_type=jnp.float32)
        + b1_ref[...], 0.0)
    o_ref[...] = (jnp.dot(t, w2_ref[...], preferred_element_type=jnp.float32)
                  + b2_ref[...])


_mesh = plsc.VectorSubcoreMesh(core_axis_name="c", subcore_axis_name="s")


@functools.partial(
    pl.kernel,
    out_type=jax.ShapeDtypeStruct((NC, N, D), jnp.bfloat16),
    mesh=_mesh,
    scratch_types=[
        pltpu.VMEM((CPT, CHUNK), jnp.int32),
        pltpu.VMEM((CPT, CHUNK), jnp.int32),
        pltpu.VMEM((CHUNK, D), jnp.bfloat16),
        pltpu.VMEM((CHUNK, D), jnp.bfloat16),
        pltpu.VMEM((CHUNK, D), jnp.bfloat16),
        pltpu.VMEM((CHUNK, D), jnp.bfloat16),
        pltpu.VMEM_SHARED((N, D), jnp.bfloat16),
        pltpu.SemaphoreType.DMA,
        pltpu.SemaphoreType.DMA,
        pltpu.SemaphoreType.DMA,
        pltpu.SemaphoreType.DMA,
    ],
    compiler_params=pltpu.CompilerParams(use_tc_tiling_on_sc=False),
)
def _segment_sum_sc(m_hbm, e3_hbm, psrc_hbm, pdst_hbm, zeros_hbm, out_hbm,
                    src_v, dst_v, r0, r1, r2, r3, z_sh, s0, s1, s2, s3):
    cid = lax.axis_index("c")
    sid = lax.axis_index("s")
    wid = sid * NC + cid
    rows = (r0, r1, r2, r3)
    sems = (s0, s1, s2, s3)

    @pl.when(wid < LAST)
    def _():
        c0 = pl.multiple_of(wid * CPT, 8)
        pltpu.sync_copy(e3_hbm.at[0, pl.ds(c0, CPT)], src_v)
        pltpu.sync_copy(e3_hbm.at[1, pl.ds(c0, CPT)], dst_v)

    @pl.when(wid == LAST)
    def _():
        pltpu.sync_copy(e3_hbm.at[0, pl.ds(_NCHUNKS_ALIGNED, MAIN_N)],
                        src_v.at[pl.ds(0, MAIN_N)])
        pltpu.sync_copy(e3_hbm.at[1, pl.ds(_NCHUNKS_ALIGNED, MAIN_N)],
                        dst_v.at[pl.ds(0, MAIN_N)])
        pltpu.sync_copy(psrc_hbm, src_v.at[pl.ds(MAIN_N, PAD_N)])
        pltpu.sync_copy(pdst_hbm, dst_v.at[pl.ds(MAIN_N, PAD_N)])

    for b in range(NBUF):
        pltpu.async_copy(m_hbm.at[src_v.at[b]], rows[b], sems[b])

    zr0 = pl.multiple_of(sid * R_STD, 8)

    @pl.when(sid < NS - 1)
    def _():
        pltpu.sync_copy(zeros_hbm.at[pl.ds(0, R_STD)],
                        z_sh.at[pl.ds(zr0, R_STD)])

    @pl.when(sid == NS - 1)
    def _():
        pltpu.sync_copy(zeros_hbm, z_sh.at[pl.ds(zr0, R_LAST)])

    plsc.subcore_barrier()

    def group(g, issue_next):
        for b in range(NBUF):
            i = g * NBUF + b
            pltpu.make_async_copy(m_hbm.at[src_v.at[i]], rows[b],
                                  sems[b]).wait()
            pltpu.sync_copy(rows[b], z_sh.at[dst_v.at[i]], add=True)
            if issue_next:
                pltpu.async_copy(m_hbm.at[src_v.at[i + NBUF]], rows[b],
                                 sems[b])

    def body(g, carry):
        group(g, True)
        return carry

    lax.fori_loop(0, GROUPS - 1, body, 0)
    group(GROUPS - 1, False)

    plsc.subcore_barrier()

    @pl.when(sid < NS - 1)
    def _():
        pltpu.sync_copy(z_sh.at[pl.ds(zr0, R_STD)],
                        out_hbm.at[cid, pl.ds(zr0, R_STD)])

    @pl.when(sid == NS - 1)
    def _():
        pltpu.sync_copy(z_sh.at[pl.ds(zr0, R_LAST)],
                        out_hbm.at[cid, pl.ds(zr0, R_LAST)])


def kernel(x, edge_index, W_pre, b_pre, W_u1, b_u1, W_u2, b_u2):
    e3 = edge_index.astype(jnp.int32).reshape(2, NCHUNKS, CHUNK)

    m = pl.pallas_call(
        _pre_body,
        out_shape=jax.ShapeDtypeStruct((M_ROWS, D), jnp.bfloat16),
    )(x, W_pre, b_pre.reshape(1, D))

    zeros = jnp.zeros((R_LAST, D), dtype=jnp.bfloat16)
    z_parts = _segment_sum_sc(m, e3, jnp.asarray(_PSRC), jnp.asarray(_PDST),
                              zeros)

    h = pl.pallas_call(
        _update_body,
        out_shape=jax.ShapeDtypeStruct((N, D), jnp.float32),
    )(x, z_parts[0], z_parts[1], W_u1[:D], W_u1[D:], b_u1.reshape(1, D),
      W_u2, b_u2.reshape(1, D))
    return h
